# baseline (speedup 1.0000x reference)
import functools

import jax
import jax.numpy as jnp
from jax import lax
from jax.experimental import pallas as pl
from jax.experimental.pallas import tpu as pltpu
from jax.experimental.pallas import tpu_sc as plsc

VOCAB = 1000000
EMB = 64
BATCH = 4096
SEQ = 50

NUM_PAIRS = BATCH // 2
NC, NS, L = 2, 16, 16
NW = NC * NS
COLS_PER_W = BATCH // NW
PAIRS_PER_W = COLS_PER_W // 2
CB = 32768
GS2 = (VOCAB + 2 * CB - 1) // (2 * CB)
VP = GS2 * 2 * CB


def _matmul_body(w2_ref, ta_ref, tb_ref, out_ref):
    ra = jnp.dot(w2_ref[...], ta_ref[...], preferred_element_type=jnp.float32)
    rb = jnp.dot(w2_ref[...], tb_ref[...], preferred_element_type=jnp.float32)
    out_ref[pl.ds(0, CB)] = ra[0]
    out_ref[pl.ds(CB, CB)] = ra[1]
    out_ref[pl.ds(2 * CB, CB)] = rb[0]
    out_ref[pl.ds(3 * CB, CB)] = rb[1]


def _token_weights(w2, table_t):
    return pl.pallas_call(
        _matmul_body,
        grid=(GS2,),
        in_specs=[
            pl.BlockSpec((2, EMB), lambda i: (0, 0)),
            pl.BlockSpec((EMB, CB), lambda i: (0, 2 * i)),
            pl.BlockSpec((EMB, CB),
                         lambda i: (0, jnp.minimum(2 * i + 1, 30))),
        ],
        out_specs=pl.BlockSpec((4 * CB,), lambda i: (i,)),
        out_shape=jax.ShapeDtypeStruct((2 * VP,), jnp.float32),
    )(w2, table_t, table_t)


def _sc_body(idxt_hbm, tw_hbm, b_hbm, out_hbm,
             idx_v, vals_v, valsb_v, colsum_v, out_v, b_v, sem):
    cid = lax.axis_index("c")
    sid = lax.axis_index("s")
    wid = sid * NC + cid
    col0 = wid * COLS_PER_W

    pltpu.sync_copy(idxt_hbm.at[:, pl.ds(col0, COLS_PER_W)], idx_v)
    pltpu.sync_copy(b_hbm, b_v)

    off = (lax.iota(jnp.int32, L) % 2) * CB
    for j in range(SEQ):
        for g in range(COLS_PER_W // L):
            sl = pl.ds(g * L, L)
            v = idx_v[j, sl]
            idx_v[j, sl] = v + (v & jnp.int32(-CB)) + off

    @pl.loop(0, SEQ)
    def _fire(j):
        pltpu.async_copy(tw_hbm.at[idx_v.at[j]], vals_v.at[j], sem)
        pltpu.async_copy(tw_hbm.at[idx_v.at[j]], valsb_v.at[j], sem)

    @pl.loop(0, SEQ)
    def _drain(j):
        pltpu.make_async_copy(tw_hbm.at[idx_v.at[j]], vals_v.at[j], sem).wait()
        pltpu.make_async_copy(tw_hbm.at[idx_v.at[j]], valsb_v.at[j], sem).wait()

    for g in range(COLS_PER_W // L):
        sl = pl.ds(g * L, L)
        acc = vals_v[0, sl]
        for j in range(1, SEQ):
            acc = acc + vals_v[j, sl]
        colsum_v[sl] = acc

    ev = lax.iota(jnp.int32, L) * 2
    od = ev + 1
    bvec = b_v[...]
    inv = jnp.float32(1.0 / SEQ)
    for m in range(PAIRS_PER_W // L):
        base = jnp.full((L,), 2 * L * m, jnp.int32)
        evens = plsc.load_gather(colsum_v, [base + ev])
        odds = plsc.load_gather(colsum_v, [base + od])
        x = (evens + odds) * inv + bvec
        out_v[pl.ds(m * L, L)] = 1.0 / (1.0 + jnp.exp(-x))

    pltpu.sync_copy(out_v, out_hbm.at[pl.ds(wid * PAIRS_PER_W, PAIRS_PER_W)])


def _gather_pool(idx_t, tw_flat, b_vec):
    mesh = plsc.VectorSubcoreMesh(core_axis_name="c", subcore_axis_name="s")
    return functools.partial(
        pl.kernel,
        out_type=jax.ShapeDtypeStruct((NUM_PAIRS,), jnp.float32),
        mesh=mesh,
        compiler_params=pltpu.CompilerParams(
            needs_layout_passes=False, use_tc_tiling_on_sc=False),
        scratch_types=[
            pltpu.VMEM((SEQ, COLS_PER_W), jnp.int32),
            pltpu.VMEM((SEQ, COLS_PER_W), jnp.float32),
            pltpu.VMEM((SEQ, COLS_PER_W), jnp.float32),
            pltpu.VMEM((COLS_PER_W,), jnp.float32),
            pltpu.VMEM((PAIRS_PER_W,), jnp.float32),
            pltpu.VMEM((L,), jnp.float32),
            pltpu.SemaphoreType.DMA,
        ],
    )(_sc_body)(idx_t, tw_flat, b_vec)


@jax.jit
def _run(indices, table, W, b):
    idx_t = indices.T.astype(jnp.int32)
    table_t = table.T
    w2 = W.reshape(2, EMB)
    tw = _token_weights(w2, table_t)
    b_vec = jnp.broadcast_to(b.astype(jnp.float32), (L,))
    out = _gather_pool(idx_t, tw, b_vec)
    return out.reshape(NUM_PAIRS, 1)


def kernel(indices, table, W, b):
    return _run(indices, table, W, b)


# speedup vs baseline: 1.0878x; 1.0878x over previous
import functools

import jax
import jax.numpy as jnp
from jax import lax
from jax.experimental import pallas as pl
from jax.experimental.pallas import tpu as pltpu
from jax.experimental.pallas import tpu_sc as plsc

VOCAB = 1000000
EMB = 64
BATCH = 4096
SEQ = 50

NUM_PAIRS = BATCH // 2
NC, NS, L = 2, 16, 16
NW = NC * NS
COLS_PER_W = BATCH // NW
PAIRS_PER_W = COLS_PER_W // 2
CB = 32768
GS = (VOCAB + CB - 1) // CB
VP = GS * CB


def _matmul_body(w2_ref, t_ref, out_ref):
    res = jnp.dot(w2_ref[...], t_ref[...],
                  preferred_element_type=jnp.float32)
    out_ref[pl.ds(0, CB)] = res[0]
    out_ref[pl.ds(CB, CB)] = res[1]


def _token_weights(w2, table_t):
    return pl.pallas_call(
        _matmul_body,
        grid=(GS,),
        in_specs=[
            pl.BlockSpec((2, EMB), lambda i: (0, 0)),
            pl.BlockSpec((EMB, CB), lambda i: (0, i)),
        ],
        out_specs=pl.BlockSpec((2 * CB,), lambda i: (i,)),
        out_shape=jax.ShapeDtypeStruct((2 * VP,), jnp.float32),
    )(w2, table_t)


def _sc_body(idxt_hbm, tw_hbm, b_hbm, out_hbm,
             idx_v, vals_v, colsum_v, out_v, b_v, sem):
    cid = lax.axis_index("c")
    sid = lax.axis_index("s")
    wid = sid * NC + cid
    col0 = wid * COLS_PER_W

    pltpu.sync_copy(idxt_hbm.at[:, pl.ds(col0, COLS_PER_W)], idx_v)
    pltpu.sync_copy(b_hbm, b_v)

    off = (lax.iota(jnp.int32, L) % 2) * CB
    for j in range(SEQ):
        for g in range(COLS_PER_W // L):
            sl = pl.ds(g * L, L)
            v = idx_v[j, sl]
            idx_v[j, sl] = v + (v & jnp.int32(-CB)) + off

    @pl.loop(0, SEQ)
    def _fire(j):
        pltpu.async_copy(tw_hbm.at[idx_v.at[j]], vals_v.at[j], sem)

    @pl.loop(0, SEQ)
    def _drain(j):
        pltpu.make_async_copy(tw_hbm.at[idx_v.at[j]], vals_v.at[j], sem).wait()

    for g in range(COLS_PER_W // L):
        sl = pl.ds(g * L, L)
        acc = vals_v[0, sl]
        for j in range(1, SEQ):
            acc = acc + vals_v[j, sl]
        colsum_v[sl] = acc

    ev = lax.iota(jnp.int32, L) * 2
    od = ev + 1
    bvec = b_v[...]
    inv = jnp.float32(1.0 / SEQ)
    for m in range(PAIRS_PER_W // L):
        base = jnp.full((L,), 2 * L * m, jnp.int32)
        evens = plsc.load_gather(colsum_v, [base + ev])
        odds = plsc.load_gather(colsum_v, [base + od])
        x = (evens + odds) * inv + bvec
        out_v[pl.ds(m * L, L)] = 1.0 / (1.0 + jnp.exp(-x))

    pltpu.sync_copy(out_v, out_hbm.at[pl.ds(wid * PAIRS_PER_W, PAIRS_PER_W)])


def _gather_pool(idx_t, tw_flat, b_vec):
    mesh = plsc.VectorSubcoreMesh(core_axis_name="c", subcore_axis_name="s")
    return functools.partial(
        pl.kernel,
        out_type=jax.ShapeDtypeStruct((NUM_PAIRS,), jnp.float32),
        mesh=mesh,
        compiler_params=pltpu.CompilerParams(
            needs_layout_passes=False, use_tc_tiling_on_sc=False),
        scratch_types=[
            pltpu.VMEM((SEQ, COLS_PER_W), jnp.int32),
            pltpu.VMEM((SEQ, COLS_PER_W), jnp.float32),
            pltpu.VMEM((COLS_PER_W,), jnp.float32),
            pltpu.VMEM((PAIRS_PER_W,), jnp.float32),
            pltpu.VMEM((L,), jnp.float32),
            pltpu.SemaphoreType.DMA,
        ],
    )(_sc_body)(idx_t, tw_flat, b_vec)


@jax.jit
def _run(indices, table, W, b):
    idx_t = indices.T.astype(jnp.int32)
    table_t = table.T
    w2 = W.reshape(2, EMB)
    tw = _token_weights(w2, table_t)
    b_vec = jnp.broadcast_to(b.astype(jnp.float32), (L,))
    out = _gather_pool(idx_t, tw, b_vec)
    return out.reshape(NUM_PAIRS, 1)


def kernel(indices, table, W, b):
    return _run(indices, table, W, b)
